# Initial kernel scaffold; baseline (speedup 1.0000x reference)
import jax
import jax.numpy as jnp
from jax.experimental import pallas as pl


def kernel(x, edge_index, edge_attr, batch, W1, b1, g1, bt1, W2, b2, g2, bt2, W3, b3, g3, bt3):
    raise NotImplementedError("write your pallas kernel here")


# speedup vs baseline: 23.6706x; 23.6706x over previous
import functools

import jax
import jax.numpy as jnp
from jax import lax
from jax.experimental import pallas as pl
from jax.experimental.pallas import tpu as pltpu
from jax.experimental.pallas import tpu_sc as plsc

N = 10000
E = 320000
H = 128
G = 64

NC = 2
NS = 16
NW = NC * NS
CHUNK = 128
CPW = 80
EPW = CPW * CHUNK
EP = NW * EPW
N_ACC = 10112
RPT = N_ACC // NS
DEGW = 16
_ZCHUNKS = (128, 128, 128, 128, 120)

@functools.cache
def _mesh():
    return plsc.VectorSubcoreMesh(core_axis_name="c", subcore_axis_name="s",
                                  num_cores=NC, num_subcores=NS)


def _zero_vmem_2d(ref, rows, width):
    per_row = width // 16

    def zb(i, carry):
        r = i // per_row
        cc = (i % per_row) * 16
        ref[r, pl.ds(cc, 16)] = jnp.zeros((16,), jnp.float32)
        return carry

    lax.fori_loop(0, rows * per_row, zb, 0)


def _deg_body(dst_hbm, out_hbm, dst_v, ones_v, zbuf_v, sem, acc):
    c = lax.axis_index("c")
    s = lax.axis_index("s")
    wid = s * NC + c

    pltpu.sync_copy(dst_hbm.at[wid], dst_v)

    def fill_ones(r, carry):
        ones_v[r, :] = jnp.ones((16,), jnp.float32)
        return carry
    lax.fori_loop(0, CHUNK, fill_ones, 0)
    _zero_vmem_2d(zbuf_v, 128, DEGW)

    base = s * RPT
    off = 0
    for sz in _ZCHUNKS:
        pltpu.sync_copy(zbuf_v.at[pl.ds(0, sz)], acc.at[pl.ds(base + off, sz)])
        off += sz
    plsc.subcore_barrier()

    def step(j, carry):
        pltpu.sync_copy(ones_v, acc.at[dst_v.at[j]], add=True)
        return carry

    lax.fori_loop(0, CPW, step, 0)
    plsc.subcore_barrier()

    pltpu.sync_copy(acc.at[pl.ds(base, RPT)], out_hbm.at[c, pl.ds(base, RPT)])


@functools.cache
def _deg_call():
    return functools.partial(
        pl.kernel,
        out_type=jax.ShapeDtypeStruct((NC, N_ACC, DEGW), jnp.float32),
        mesh=_mesh(),
        scratch_types=[
            pltpu.VMEM((CPW, CHUNK), jnp.int32),
            pltpu.VMEM((CHUNK, DEGW), jnp.float32),
            pltpu.VMEM((128, DEGW), jnp.float32),
            pltpu.SemaphoreType.DMA,
            pltpu.VMEM_SHARED((N_ACC, DEGW), jnp.float32),
        ],
    )(_deg_body)


HALF = CPW // 2


def _edge_body(hs_hbm, src_hbm, dst_hbm, out_hbm,
               src_v, dst_v, rows0, rows1, sem0, sem1, acc):
    c = lax.axis_index("c")
    s = lax.axis_index("s")
    wid = s * NC + c

    _zero_vmem_2d(rows0, CHUNK, H)
    base = s * RPT
    off = 0
    for sz in _ZCHUNKS:
        pltpu.sync_copy(rows0.at[pl.ds(0, sz)], acc.at[pl.ds(base + off, sz)])
        off += sz
    plsc.subcore_barrier()

    for half in range(CPW // HALF):
        pltpu.sync_copy(src_hbm.at[wid, pl.ds(half * HALF, HALF)], src_v)
        pltpu.sync_copy(dst_hbm.at[wid, pl.ds(half * HALF, HALF)], dst_v)

        pltpu.async_copy(hs_hbm.at[src_v.at[0]], rows0, sem0)

        def pair(i, carry):
            j = 2 * i
            pltpu.make_async_copy(hs_hbm.at[src_v.at[j]], rows0, sem0).wait()
            pltpu.async_copy(hs_hbm.at[src_v.at[j + 1]], rows1, sem1)
            pltpu.sync_copy(rows0, acc.at[dst_v.at[j]], add=True)
            pltpu.make_async_copy(hs_hbm.at[src_v.at[j + 1]], rows1,
                                  sem1).wait()

            @pl.when(i < HALF // 2 - 1)
            def _():
                pltpu.async_copy(hs_hbm.at[src_v.at[j + 2]], rows0, sem0)
            pltpu.sync_copy(rows1, acc.at[dst_v.at[j + 1]], add=True)
            return carry

        lax.fori_loop(0, HALF // 2, pair, 0)

    plsc.subcore_barrier()
    pltpu.sync_copy(acc.at[pl.ds(base, RPT)], out_hbm.at[c, pl.ds(base, RPT)])


@functools.cache
def _edge_call():
    return functools.partial(
        pl.kernel,
        out_type=jax.ShapeDtypeStruct((NC, N_ACC, H), jnp.float32),
        mesh=_mesh(),
        scratch_types=[
            pltpu.VMEM((HALF, CHUNK), jnp.int32),
            pltpu.VMEM((HALF, CHUNK), jnp.int32),
            pltpu.VMEM((CHUNK, H), jnp.float32),
            pltpu.VMEM((CHUNK, H), jnp.float32),
            pltpu.SemaphoreType.DMA,
            pltpu.SemaphoreType.DMA,
            pltpu.VMEM_SHARED((N_ACC, H), jnp.float32),
        ],
    )(_edge_body)


def _dot(a, b):
    return lax.dot_general(a, b, (((1,), (0,)), ((), ())),
                           precision=lax.Precision.HIGHEST,
                           preferred_element_type=jnp.float32)


def _tcd_body(degp_ref, o_ref):
    dp = degp_ref[0] + degp_ref[1]
    deg = jnp.sum(dp, axis=1, keepdims=True) * (1.0 / DEGW) + 1.0
    dinv = lax.rsqrt(deg)
    o_ref[...] = jnp.broadcast_to(dinv[:N], (N, H))


_tcd_call = pl.pallas_call(
    _tcd_body, out_shape=jax.ShapeDtypeStruct((N, H), jnp.float32))


def _tc0_body(x_ref, w_ref, dinv_ref, o_ref):
    o_ref[...] = _dot(x_ref[...], w_ref[...]) * dinv_ref[...]


_tc0_call = pl.pallas_call(
    _tc0_body, out_shape=jax.ShapeDtypeStruct((N, H), jnp.float32))


def _bn_sigmoid(p_ref, hs_ref, dinv_ref, b_ref, g_ref, bt_ref):
    agg = (p_ref[0, :N] + p_ref[1, :N] + hs_ref[...]) * dinv_ref[...]
    z = agg + b_ref[...][None, :]
    m = jnp.mean(z, axis=0, keepdims=True)
    zc = z - m
    v = jnp.mean(zc * zc, axis=0, keepdims=True)
    zn = g_ref[...][None, :] * zc * lax.rsqrt(v + 1e-5) + bt_ref[...][None, :]
    return jax.nn.sigmoid(zn)


def _tcmid_body(p_ref, hs_ref, dinv_ref, b_ref, g_ref, bt_ref, w_ref, o_ref):
    a = _bn_sigmoid(p_ref, hs_ref, dinv_ref, b_ref, g_ref, bt_ref)
    o_ref[...] = _dot(a, w_ref[...]) * dinv_ref[...]


_tcmid_call = pl.pallas_call(
    _tcmid_body, out_shape=jax.ShapeDtypeStruct((N, H), jnp.float32))


def _tcfin_body(p_ref, hs_ref, dinv_ref, b_ref, g_ref, bt_ref, batch_ref,
                o_ref):
    a = _bn_sigmoid(p_ref, hs_ref, dinv_ref, b_ref, g_ref, bt_ref)
    ids = batch_ref[...]
    io = lax.broadcasted_iota(jnp.int32, (G, N), 0)
    oh = (io == ids).astype(jnp.float32)
    sums = _dot(oh, a)
    cnt = jnp.sum(oh, axis=1, keepdims=True)
    o_ref[...] = sums / jnp.maximum(cnt, 1.0)


_tcfin_call = pl.pallas_call(
    _tcfin_body, out_shape=jax.ShapeDtypeStruct((G, H), jnp.float32))


def kernel(x, edge_index, edge_attr, batch,
           W1, b1, g1, bt1, W2, b2, g2, bt2, W3, b3, g3, bt3):
    src = edge_index[0]
    dst = edge_index[1]
    pad = EP - E
    ar = jnp.arange(pad, dtype=jnp.int32)
    srcp = jnp.concatenate([src, ar % N]).reshape(NW, CPW, CHUNK)
    dstp = jnp.concatenate([dst, N + ar % (N_ACC - N)]).reshape(NW, CPW, CHUNK)

    degp = _deg_call()(dstp)
    dinvb = _tcd_call(degp)
    hs1 = _tc0_call(x, W1, dinvb)
    p1 = _edge_call()(hs1, srcp, dstp)
    hs2 = _tcmid_call(p1, hs1, dinvb, b1, g1, bt1, W2)
    p2 = _edge_call()(hs2, srcp, dstp)
    hs3 = _tcmid_call(p2, hs2, dinvb, b2, g2, bt2, W3)
    p3 = _edge_call()(hs3, srcp, dstp)
    return _tcfin_call(p3, hs3, dinvb, b3, g3, bt3, batch.reshape(1, N))


# speedup vs baseline: 24.0063x; 1.0142x over previous
import functools

import jax
import jax.numpy as jnp
from jax import lax
from jax.experimental import pallas as pl
from jax.experimental.pallas import tpu as pltpu
from jax.experimental.pallas import tpu_sc as plsc

N = 10000
E = 320000
H = 128
G = 64

NC = 2
NS = 16
NW = NC * NS
CHUNK = 128
CPW = 80
EPW = CPW * CHUNK
EP = NW * EPW
N_ACC = 10112
RPT = N_ACC // NS
DEGW = 16
_ZCHUNKS = (128, 128, 128, 128, 120)

@functools.cache
def _mesh():
    return plsc.VectorSubcoreMesh(core_axis_name="c", subcore_axis_name="s",
                                  num_cores=NC, num_subcores=NS)


def _zero_vmem_2d(ref, rows, width):
    per_row = width // 16

    def zb(i, carry):
        r = i // per_row
        cc = (i % per_row) * 16
        ref[r, pl.ds(cc, 16)] = jnp.zeros((16,), jnp.float32)
        return carry

    lax.fori_loop(0, rows * per_row, zb, 0)


def _deg_body(dst_hbm, out_hbm, dst_v, ones_v, zbuf_v, sem, acc):
    c = lax.axis_index("c")
    s = lax.axis_index("s")
    wid = s * NC + c

    pltpu.sync_copy(dst_hbm.at[wid], dst_v)

    def fill_ones(r, carry):
        ones_v[r, :] = jnp.ones((16,), jnp.float32)
        return carry
    lax.fori_loop(0, CHUNK, fill_ones, 0)
    _zero_vmem_2d(zbuf_v, 128, DEGW)

    base = s * RPT
    off = 0
    for sz in _ZCHUNKS:
        pltpu.sync_copy(zbuf_v.at[pl.ds(0, sz)], acc.at[pl.ds(base + off, sz)])
        off += sz
    plsc.subcore_barrier()

    def step(j, carry):
        pltpu.sync_copy(ones_v, acc.at[dst_v.at[j]], add=True)
        return carry

    lax.fori_loop(0, CPW, step, 0)
    plsc.subcore_barrier()

    pltpu.sync_copy(acc.at[pl.ds(base, RPT)], out_hbm.at[c, pl.ds(base, RPT)])


@functools.cache
def _deg_call():
    return functools.partial(
        pl.kernel,
        out_type=jax.ShapeDtypeStruct((NC, N_ACC, DEGW), jnp.float32),
        mesh=_mesh(),
        scratch_types=[
            pltpu.VMEM((CPW, CHUNK), jnp.int32),
            pltpu.VMEM((CHUNK, DEGW), jnp.float32),
            pltpu.VMEM((128, DEGW), jnp.float32),
            pltpu.SemaphoreType.DMA,
            pltpu.VMEM_SHARED((N_ACC, DEGW), jnp.float32),
        ],
    )(_deg_body)


HALF = CPW // 2


def _edge_body(hs_hbm, src_hbm, dst_hbm, out_hbm,
               src_v, dst_v, rows0, rows1, sem0, sem1, acc):
    c = lax.axis_index("c")
    s = lax.axis_index("s")
    wid = s * NC + c

    _zero_vmem_2d(rows0, CHUNK, H)
    base = s * RPT
    off = 0
    for sz in _ZCHUNKS:
        pltpu.sync_copy(rows0.at[pl.ds(0, sz)], acc.at[pl.ds(base + off, sz)])
        off += sz
    plsc.subcore_barrier()

    for half in range(CPW // HALF):
        pltpu.sync_copy(src_hbm.at[wid, pl.ds(half * HALF, HALF)], src_v)
        pltpu.sync_copy(dst_hbm.at[wid, pl.ds(half * HALF, HALF)], dst_v)

        pltpu.async_copy(hs_hbm.at[src_v.at[0]], rows0, sem0)

        def pair(i, carry):
            j = 2 * i
            pltpu.make_async_copy(hs_hbm.at[src_v.at[j]], rows0, sem0).wait()
            pltpu.async_copy(hs_hbm.at[src_v.at[j + 1]], rows1, sem1)
            pltpu.sync_copy(rows0, acc.at[dst_v.at[j]], add=True)
            pltpu.make_async_copy(hs_hbm.at[src_v.at[j + 1]], rows1,
                                  sem1).wait()

            @pl.when(i < HALF // 2 - 1)
            def _():
                pltpu.async_copy(hs_hbm.at[src_v.at[j + 2]], rows0, sem0)
            pltpu.sync_copy(rows1, acc.at[dst_v.at[j + 1]], add=True)
            return carry

        lax.fori_loop(0, HALF // 2, pair, 0)

    plsc.subcore_barrier()
    pltpu.sync_copy(acc.at[pl.ds(base, RPT)], out_hbm.at[c, pl.ds(base, RPT)])


@functools.cache
def _edge_call():
    return functools.partial(
        pl.kernel,
        out_type=jax.ShapeDtypeStruct((NC, N_ACC, H), jnp.float32),
        mesh=_mesh(),
        scratch_types=[
            pltpu.VMEM((HALF, CHUNK), jnp.int32),
            pltpu.VMEM((HALF, CHUNK), jnp.int32),
            pltpu.VMEM((CHUNK, H), jnp.float32),
            pltpu.VMEM((CHUNK, H), jnp.float32),
            pltpu.SemaphoreType.DMA,
            pltpu.SemaphoreType.DMA,
            pltpu.VMEM_SHARED((N_ACC, H), jnp.float32),
        ],
    )(_edge_body)


def _dot(a, b):
    return lax.dot_general(a, b, (((1,), (0,)), ((), ())),
                           precision=lax.Precision.HIGHEST,
                           preferred_element_type=jnp.float32)


def _tcd_body(degp_ref, h1_ref, dinv_ref, hs1_ref):
    dp = degp_ref[0] + degp_ref[1]
    deg = jnp.sum(dp, axis=1, keepdims=True) * (1.0 / DEGW) + 1.0
    dinv = jnp.broadcast_to(lax.rsqrt(deg)[:N], (N, H))
    dinv_ref[...] = dinv
    hs1_ref[...] = h1_ref[...] * dinv


_tcd_call = pl.pallas_call(
    _tcd_body, out_shape=[jax.ShapeDtypeStruct((N, H), jnp.float32),
                          jax.ShapeDtypeStruct((N, H), jnp.float32)])


def _tc0_body(x_ref, w_ref, o_ref):
    o_ref[...] = _dot(x_ref[...], w_ref[...])


_tc0_call = pl.pallas_call(
    _tc0_body, out_shape=jax.ShapeDtypeStruct((N, H), jnp.float32))


def _bn_sigmoid(p_ref, hs_ref, dinv_ref, b_ref, g_ref, bt_ref):
    agg = (p_ref[0, :N] + p_ref[1, :N] + hs_ref[...]) * dinv_ref[...]
    z = agg + b_ref[...][None, :]
    m = jnp.mean(z, axis=0, keepdims=True)
    zc = z - m
    v = jnp.mean(zc * zc, axis=0, keepdims=True)
    zn = g_ref[...][None, :] * zc * lax.rsqrt(v + 1e-5) + bt_ref[...][None, :]
    return jax.nn.sigmoid(zn)


def _tcmid_body(p_ref, hs_ref, dinv_ref, b_ref, g_ref, bt_ref, w_ref, o_ref):
    a = _bn_sigmoid(p_ref, hs_ref, dinv_ref, b_ref, g_ref, bt_ref)
    o_ref[...] = _dot(a, w_ref[...]) * dinv_ref[...]


_tcmid_call = pl.pallas_call(
    _tcmid_body, out_shape=jax.ShapeDtypeStruct((N, H), jnp.float32))


def _tcfin_body(p_ref, hs_ref, dinv_ref, b_ref, g_ref, bt_ref, batch_ref,
                o_ref):
    a = _bn_sigmoid(p_ref, hs_ref, dinv_ref, b_ref, g_ref, bt_ref)
    ids = batch_ref[...]
    io = lax.broadcasted_iota(jnp.int32, (G, N), 0)
    oh = (io == ids).astype(jnp.float32)
    sums = _dot(oh, a)
    cnt = jnp.sum(oh, axis=1, keepdims=True)
    o_ref[...] = sums / jnp.maximum(cnt, 1.0)


_tcfin_call = pl.pallas_call(
    _tcfin_body, out_shape=jax.ShapeDtypeStruct((G, H), jnp.float32))


def kernel(x, edge_index, edge_attr, batch,
           W1, b1, g1, bt1, W2, b2, g2, bt2, W3, b3, g3, bt3):
    src = edge_index[0]
    dst = edge_index[1]
    pad = EP - E
    ar = jnp.arange(pad, dtype=jnp.int32)
    srcp = jnp.concatenate([src, ar % N]).reshape(NW, CPW, CHUNK)
    dstp = jnp.concatenate([dst, N + ar % (N_ACC - N)]).reshape(NW, CPW, CHUNK)

    h1 = _tc0_call(x, W1)
    degp = _deg_call()(dstp)
    dinvb, hs1 = _tcd_call(degp, h1)
    p1 = _edge_call()(hs1, srcp, dstp)
    hs2 = _tcmid_call(p1, hs1, dinvb, b1, g1, bt1, W2)
    p2 = _edge_call()(hs2, srcp, dstp)
    hs3 = _tcmid_call(p2, hs2, dinvb, b2, g2, bt2, W3)
    p3 = _edge_call()(hs3, srcp, dstp)
    return _tcfin_call(p3, hs3, dinvb, b3, g3, bt3, batch.reshape(1, N))
